# Initial kernel scaffold; baseline (speedup 1.0000x reference)
import jax
import jax.numpy as jnp
from jax.experimental import pallas as pl


def kernel(x, edge_index, W_in, b_in, T, raw_w, alpha, ln_g, ln_b):
    raise NotImplementedError("write your pallas kernel here")


# speedup vs baseline: 1.4452x; 1.4452x over previous
import functools

import jax
import jax.numpy as jnp
from jax import lax
from jax.experimental import pallas as pl
from jax.experimental.pallas import tpu as pltpu
from jax.experimental.pallas import tpu_sc as plsc

N_NODES = 10000
N_EDGES = 160000
IN_DIM = 128
D = 16

NC = 2
NS = 16
NW = NC * NS
CHUNK = 128
N_CHUNKS = N_EDGES // CHUNK
N_ROW_CHUNKS = N_NODES // CHUNK
ROW_TAIL = N_NODES - N_ROW_CHUNKS * CHUNK


def _proj_body(x_ref, w_ref, b_ref, o_ref):
    h = jnp.dot(x_ref[...], w_ref[...], preferred_element_type=jnp.float32)
    o_ref[...] = jnp.maximum(h + b_ref[...], 0.0)


def _project(x, W_in, b_in):
    return pl.pallas_call(
        _proj_body,
        grid=(10,),
        in_specs=[
            pl.BlockSpec((N_NODES // 10, IN_DIM), lambda i: (i, 0)),
            pl.BlockSpec((IN_DIM, D), lambda i: (0, 0)),
            pl.BlockSpec((1, D), lambda i: (0, 0)),
        ],
        out_specs=pl.BlockSpec((N_NODES // 10, D), lambda i: (i, 0)),
        out_shape=jax.ShapeDtypeStruct((N_NODES, D), jnp.float32),
    )(x, W_in, b_in.reshape(1, D))


def _softplus_body(r_ref, a_ref, o_ref):
    r = r_ref[...]
    sp = jnp.maximum(r, 0.0) + jnp.log1p(jnp.exp(-jnp.abs(r)))
    o_ref[...] = a_ref[0, 0] * sp


def _edge_weights(raw_w, alpha):
    w2 = pl.pallas_call(
        _softplus_body,
        in_specs=[
            pl.BlockSpec((2 * N_EDGES // 128, 128), lambda: (0, 0)),
            pl.BlockSpec((1, 1), lambda: (0, 0)),
        ],
        out_specs=pl.BlockSpec((2 * N_EDGES // 128, 128), lambda: (0, 0)),
        out_shape=jax.ShapeDtypeStruct((2 * N_EDGES // 128, 128), jnp.float32),
    )(raw_w.reshape(2 * N_EDGES // 128, 128), jnp.asarray(alpha, jnp.float32).reshape(1, 1))
    return w2.reshape(2 * N_EDGES)


_GDN = lax.GatherDimensionNumbers(
    offset_dims=(), collapsed_slice_dims=(0,), start_index_map=(0,)
)


def _vreg_gather(x, idx):
    return lax.gather(x, idx[:, None], _GDN, (1,),
                      mode=lax.GatherScatterMode.PROMISE_IN_BOUNDS)


_MESH = plsc.VectorSubcoreMesh(
    core_axis_name="c", subcore_axis_name="s", num_cores=NC, num_subcores=NS
)


@functools.partial(
    pl.kernel,
    out_type=jax.ShapeDtypeStruct((NC, N_NODES, 2 * D), jnp.float32),
    mesh=_MESH,
    compiler_params=pltpu.CompilerParams(
        needs_layout_passes=False, use_tc_tiling_on_sc=False
    ),
    scratch_types=[
        pltpu.VMEM((CHUNK,), jnp.int32),
        pltpu.VMEM((CHUNK,), jnp.int32),
        pltpu.VMEM((CHUNK, D), jnp.float32),
        pltpu.VMEM((CHUNK, D), jnp.float32),
        pltpu.VMEM((CHUNK * D * D,), jnp.float32),
        pltpu.VMEM((CHUNK,), jnp.float32),
        pltpu.VMEM((CHUNK, 2 * D), jnp.float32),
        pltpu.SemaphoreType.DMA,
        pltpu.VMEM_SHARED((N_NODES, 2 * D), jnp.float32),
    ],
)
def _edge_kernel(h_hbm, src_hbm, dst_hbm, tf_hbm, tr_hbm, wf_hbm, wr_hbm,
                 out_hbm, sidx, didx, hsrc, hdst, tbuf, wbuf, msg, sem, acc):
    cid = lax.axis_index("c")
    sid = lax.axis_index("s")
    wid = sid * NC + cid

    zero16 = jnp.zeros((D,), jnp.float32)
    lanes = lax.iota(jnp.int32, D)
    one_hot = jnp.where(lanes == 0, 1.0, 0.0).astype(jnp.float32)

    def _zrow(c, carry):
        msg[c, pl.ds(0, D)] = zero16
        msg[c, pl.ds(D, D)] = zero16
        return carry

    lax.fori_loop(0, CHUNK, _zrow, 0)
    n_rc = jnp.where(sid < N_ROW_CHUNKS - (N_ROW_CHUNKS // NS) * NS,
                     N_ROW_CHUNKS // NS + 1, N_ROW_CHUNKS // NS)

    def _zchunk(t, carry):
        start = (sid + NS * t) * CHUNK
        pltpu.sync_copy(msg.at[pl.ds(0, CHUNK)], acc.at[pl.ds(start, CHUNK)])
        return carry

    lax.fori_loop(0, n_rc, _zchunk, 0)

    @pl.when(sid == NS - 1)
    def _ztail():
        pltpu.sync_copy(msg.at[pl.ds(0, ROW_TAIL)],
                        acc.at[pl.ds(N_ROW_CHUNKS * CHUNK, ROW_TAIL)])

    plsc.subcore_barrier()

    def _setrow(c, carry):
        msg[c, pl.ds(D, D)] = one_hot
        return carry

    lax.fori_loop(0, CHUNK, _setrow, 0)

    lanes16 = lanes * D
    _K_IDX = [jnp.full((D,), k, jnp.int32) for k in range(D)]

    def _direction(t_hbm, w_hbm, xin, xsub, scat_idx, base):
        pltpu.sync_copy(t_hbm.at[pl.ds(base * D * D, CHUNK * D * D)], tbuf)
        pltpu.sync_copy(w_hbm.at[pl.ds(base, CHUNK)], wbuf)

        def _edge(c, carry):
            cvec = jnp.full((D,), c, jnp.int32)
            tbase = lanes16 + c * (D * D)
            xs = xin[c, :]
            m = zero16
            for k in range(D):
                col = plsc.load_gather(tbuf, [tbase + k])
                xk = _vreg_gather(xs, _K_IDX[k])
                m = m + col * xk
            m = m - xsub[c, :]
            wv = plsc.load_gather(wbuf, [cvec])
            msg[c, pl.ds(0, D)] = wv * m
            return carry

        lax.fori_loop(0, CHUNK, _edge, 0)
        pltpu.sync_copy(msg, acc.at[scat_idx], add=True)

    n_chunks = jnp.where(wid < N_CHUNKS - (N_CHUNKS // NW) * NW, N_CHUNKS // NW + 1,
                         N_CHUNKS // NW)

    def _chunk(j, carry):
        base = (wid + j * NW) * CHUNK
        pltpu.sync_copy(src_hbm.at[pl.ds(base, CHUNK)], sidx)
        pltpu.sync_copy(dst_hbm.at[pl.ds(base, CHUNK)], didx)
        pltpu.async_copy(h_hbm.at[sidx], hsrc, sem).wait()
        pltpu.async_copy(h_hbm.at[didx], hdst, sem).wait()
        _direction(tf_hbm, wf_hbm, hsrc, hdst, didx, base)
        _direction(tr_hbm, wr_hbm, hdst, hsrc, sidx, base)
        return carry

    lax.fori_loop(0, n_chunks, _chunk, 0)
    plsc.subcore_barrier()

    def _dchunk(t, carry):
        start = (sid + NS * t) * CHUNK
        pltpu.sync_copy(acc.at[pl.ds(start, CHUNK)],
                        out_hbm.at[cid, pl.ds(start, CHUNK)])
        return carry

    lax.fori_loop(0, n_rc, _dchunk, 0)

    @pl.when(sid == NS - 1)
    def _dtail():
        pltpu.sync_copy(acc.at[pl.ds(N_ROW_CHUNKS * CHUNK, ROW_TAIL)],
                        out_hbm.at[cid, pl.ds(N_ROW_CHUNKS * CHUNK, ROW_TAIL)])


def _final_body(h_ref, a0_ref, a1_ref, g_ref, b_ref, o_ref):
    accs = a0_ref[...] + a1_ref[...]
    msgv = accs[:, 0:D]
    deg = accs[:, D:D + 1]
    out = h_ref[...] + msgv / jnp.maximum(deg, 1.0)
    mu = jnp.mean(out, axis=-1, keepdims=True)
    var = jnp.mean((out - mu) ** 2, axis=-1, keepdims=True)
    out = (out - mu) * lax.rsqrt(var + 1e-5) * g_ref[...] + b_ref[...]
    o_ref[...] = jnp.maximum(out, 0.0)


def _finalize(h, acc, ln_g, ln_b):
    blk = N_NODES // 10
    return pl.pallas_call(
        _final_body,
        grid=(10,),
        in_specs=[
            pl.BlockSpec((blk, D), lambda i: (i, 0)),
            pl.BlockSpec((blk, 2 * D), lambda i: (i, 0)),
            pl.BlockSpec((blk, 2 * D), lambda i: (i, 0)),
            pl.BlockSpec((1, D), lambda i: (0, 0)),
            pl.BlockSpec((1, D), lambda i: (0, 0)),
        ],
        out_specs=pl.BlockSpec((blk, D), lambda i: (i, 0)),
        out_shape=jax.ShapeDtypeStruct((N_NODES, D), jnp.float32),
    )(h, acc[0], acc[1], ln_g.reshape(1, D), ln_b.reshape(1, D))


def kernel(x, edge_index, W_in, b_in, T, raw_w, alpha, ln_g, ln_b):
    src = edge_index[0]
    dst = edge_index[1]
    h = _project(x, W_in, b_in)
    w_all = _edge_weights(raw_w, alpha)
    tf = T[:N_EDGES].reshape(N_EDGES * D * D)
    tr = T[N_EDGES:].reshape(N_EDGES * D * D)
    acc = _edge_kernel(h, src, dst, tf, tr, w_all[:N_EDGES], w_all[N_EDGES:])
    return _finalize(h, acc, ln_g, ln_b)


# speedup vs baseline: 1.5565x; 1.0771x over previous
import functools

import jax
import jax.numpy as jnp
from jax import lax
from jax.experimental import pallas as pl
from jax.experimental.pallas import tpu as pltpu
from jax.experimental.pallas import tpu_sc as plsc

N_NODES = 10000
N_EDGES = 160000
IN_DIM = 128
D = 16

NC = 2
NS = 16
NW = NC * NS
CHUNK = 128
N_CHUNKS = N_EDGES // CHUNK
N_ROW_CHUNKS = N_NODES // CHUNK
ROW_TAIL = N_NODES - N_ROW_CHUNKS * CHUNK


def _proj_body(x_ref, w_ref, b_ref, o_ref):
    h = jnp.dot(x_ref[...], w_ref[...], preferred_element_type=jnp.float32)
    o_ref[...] = jnp.maximum(h + b_ref[...], 0.0)


def _project(x, W_in, b_in):
    return pl.pallas_call(
        _proj_body,
        grid=(10,),
        in_specs=[
            pl.BlockSpec((N_NODES // 10, IN_DIM), lambda i: (i, 0)),
            pl.BlockSpec((IN_DIM, D), lambda i: (0, 0)),
            pl.BlockSpec((1, D), lambda i: (0, 0)),
        ],
        out_specs=pl.BlockSpec((N_NODES // 10, D), lambda i: (i, 0)),
        out_shape=jax.ShapeDtypeStruct((N_NODES, D), jnp.float32),
    )(x, W_in, b_in.reshape(1, D))


def _softplus_body(r_ref, a_ref, o_ref):
    r = r_ref[...]
    sp = jnp.maximum(r, 0.0) + jnp.log1p(jnp.exp(-jnp.abs(r)))
    o_ref[...] = a_ref[0, 0] * sp


def _edge_weights(raw_w, alpha):
    w2 = pl.pallas_call(
        _softplus_body,
        in_specs=[
            pl.BlockSpec((2 * N_EDGES // 128, 128), lambda: (0, 0)),
            pl.BlockSpec((1, 1), lambda: (0, 0)),
        ],
        out_specs=pl.BlockSpec((2 * N_EDGES // 128, 128), lambda: (0, 0)),
        out_shape=jax.ShapeDtypeStruct((2 * N_EDGES // 128, 128), jnp.float32),
    )(raw_w.reshape(2 * N_EDGES // 128, 128), jnp.asarray(alpha, jnp.float32).reshape(1, 1))
    return w2.reshape(2 * N_EDGES)


_GDN = lax.GatherDimensionNumbers(
    offset_dims=(), collapsed_slice_dims=(0,), start_index_map=(0,)
)


def _vreg_gather(x, idx):
    return lax.gather(x, idx[:, None], _GDN, (1,),
                      mode=lax.GatherScatterMode.PROMISE_IN_BOUNDS)


_MESH = plsc.VectorSubcoreMesh(
    core_axis_name="c", subcore_axis_name="s", num_cores=NC, num_subcores=NS
)


@functools.partial(
    pl.kernel,
    out_type=jax.ShapeDtypeStruct((NC, N_NODES, 2 * D), jnp.float32),
    mesh=_MESH,
    compiler_params=pltpu.CompilerParams(
        needs_layout_passes=False, use_tc_tiling_on_sc=False
    ),
    scratch_types=[
        pltpu.VMEM((CHUNK,), jnp.int32),
        pltpu.VMEM((CHUNK,), jnp.int32),
        pltpu.VMEM((CHUNK, D), jnp.float32),
        pltpu.VMEM((CHUNK, D), jnp.float32),
        pltpu.VMEM((CHUNK * D * D,), jnp.float32),
        pltpu.VMEM((CHUNK,), jnp.float32),
        pltpu.VMEM((CHUNK, 2 * D), jnp.float32),
        pltpu.SemaphoreType.DMA,
        pltpu.VMEM_SHARED((N_NODES, 2 * D), jnp.float32),
    ],
)
def _edge_kernel(h_hbm, src_hbm, dst_hbm, tf_hbm, tr_hbm, wf_hbm, wr_hbm,
                 out_hbm, sidx, didx, hsrc, hdst, tbuf, wbuf, msg, sem, acc):
    cid = lax.axis_index("c")
    sid = lax.axis_index("s")
    wid = sid * NC + cid

    zero16 = jnp.zeros((D,), jnp.float32)
    lanes = lax.iota(jnp.int32, D)
    one_hot = jnp.where(lanes == 0, 1.0, 0.0).astype(jnp.float32)

    def _zrow(c, carry):
        msg[c, pl.ds(0, D)] = zero16
        msg[c, pl.ds(D, D)] = zero16
        return carry

    lax.fori_loop(0, CHUNK, _zrow, 0)
    n_rc = jnp.where(sid < N_ROW_CHUNKS - (N_ROW_CHUNKS // NS) * NS,
                     N_ROW_CHUNKS // NS + 1, N_ROW_CHUNKS // NS)

    def _zchunk(t, carry):
        start = (sid + NS * t) * CHUNK
        pltpu.sync_copy(msg.at[pl.ds(0, CHUNK)], acc.at[pl.ds(start, CHUNK)])
        return carry

    lax.fori_loop(0, n_rc, _zchunk, 0)

    @pl.when(sid == NS - 1)
    def _ztail():
        pltpu.sync_copy(msg.at[pl.ds(0, ROW_TAIL)],
                        acc.at[pl.ds(N_ROW_CHUNKS * CHUNK, ROW_TAIL)])

    plsc.subcore_barrier()

    def _setrow(c, carry):
        msg[c, pl.ds(D, D)] = one_hot
        return carry

    lax.fori_loop(0, CHUNK, _setrow, 0)

    lanes16 = lanes * D
    _ROT = [(lanes + j) % D for j in range(D)]
    _DIAG = [lanes16 + _ROT[j] for j in range(D)]

    def _direction(t_hbm, w_hbm, xin, xsub, scat_idx, base):
        pltpu.sync_copy(t_hbm.at[pl.ds(base * D * D, CHUNK * D * D)], tbuf)
        pltpu.sync_copy(w_hbm.at[pl.ds(base, CHUNK)], wbuf)

        def _edge(c, carry):
            cvec = jnp.full((D,), c, jnp.int32)
            cbase = jnp.full((D,), c * (D * D), jnp.int32)
            xs = xin[c, :]
            acc4 = [zero16, zero16, zero16, zero16]
            for j in range(D):
                col = plsc.load_gather(tbuf, [_DIAG[j] + cbase])
                xr = _vreg_gather(xs, _ROT[j])
                acc4[j % 4] = acc4[j % 4] + col * xr
            m = (acc4[0] + acc4[1]) + (acc4[2] + acc4[3]) - xsub[c, :]
            wv = plsc.load_gather(wbuf, [cvec])
            msg[c, pl.ds(0, D)] = wv * m
            return carry

        lax.fori_loop(0, CHUNK, _edge, 0)
        pltpu.sync_copy(msg, acc.at[scat_idx], add=True)

    n_chunks = jnp.where(wid < N_CHUNKS - (N_CHUNKS // NW) * NW, N_CHUNKS // NW + 1,
                         N_CHUNKS // NW)

    def _chunk(j, carry):
        base = (wid + j * NW) * CHUNK
        pltpu.sync_copy(src_hbm.at[pl.ds(base, CHUNK)], sidx)
        pltpu.sync_copy(dst_hbm.at[pl.ds(base, CHUNK)], didx)
        pltpu.async_copy(h_hbm.at[sidx], hsrc, sem).wait()
        pltpu.async_copy(h_hbm.at[didx], hdst, sem).wait()
        _direction(tf_hbm, wf_hbm, hsrc, hdst, didx, base)
        _direction(tr_hbm, wr_hbm, hdst, hsrc, sidx, base)
        return carry

    lax.fori_loop(0, n_chunks, _chunk, 0)
    plsc.subcore_barrier()

    def _dchunk(t, carry):
        start = (sid + NS * t) * CHUNK
        pltpu.sync_copy(acc.at[pl.ds(start, CHUNK)],
                        out_hbm.at[cid, pl.ds(start, CHUNK)])
        return carry

    lax.fori_loop(0, n_rc, _dchunk, 0)

    @pl.when(sid == NS - 1)
    def _dtail():
        pltpu.sync_copy(acc.at[pl.ds(N_ROW_CHUNKS * CHUNK, ROW_TAIL)],
                        out_hbm.at[cid, pl.ds(N_ROW_CHUNKS * CHUNK, ROW_TAIL)])


def _final_body(h_ref, a0_ref, a1_ref, g_ref, b_ref, o_ref):
    accs = a0_ref[...] + a1_ref[...]
    msgv = accs[:, 0:D]
    deg = accs[:, D:D + 1]
    out = h_ref[...] + msgv / jnp.maximum(deg, 1.0)
    mu = jnp.mean(out, axis=-1, keepdims=True)
    var = jnp.mean((out - mu) ** 2, axis=-1, keepdims=True)
    out = (out - mu) * lax.rsqrt(var + 1e-5) * g_ref[...] + b_ref[...]
    o_ref[...] = jnp.maximum(out, 0.0)


def _finalize(h, acc, ln_g, ln_b):
    blk = N_NODES // 10
    return pl.pallas_call(
        _final_body,
        grid=(10,),
        in_specs=[
            pl.BlockSpec((blk, D), lambda i: (i, 0)),
            pl.BlockSpec((blk, 2 * D), lambda i: (i, 0)),
            pl.BlockSpec((blk, 2 * D), lambda i: (i, 0)),
            pl.BlockSpec((1, D), lambda i: (0, 0)),
            pl.BlockSpec((1, D), lambda i: (0, 0)),
        ],
        out_specs=pl.BlockSpec((blk, D), lambda i: (i, 0)),
        out_shape=jax.ShapeDtypeStruct((N_NODES, D), jnp.float32),
    )(h, acc[0], acc[1], ln_g.reshape(1, D), ln_b.reshape(1, D))


def kernel(x, edge_index, W_in, b_in, T, raw_w, alpha, ln_g, ln_b):
    src = edge_index[0]
    dst = edge_index[1]
    h = _project(x, W_in, b_in)
    w_all = _edge_weights(raw_w, alpha)
    tf = T[:N_EDGES].reshape(N_EDGES * D * D)
    tr = T[N_EDGES:].reshape(N_EDGES * D * D)
    acc = _edge_kernel(h, src, dst, tf, tr, w_all[:N_EDGES], w_all[N_EDGES:])
    return _finalize(h, acc, ln_g, ln_b)


# speedup vs baseline: 2.0265x; 1.3019x over previous
import functools

import jax
import jax.numpy as jnp
from jax import lax
from jax.experimental import pallas as pl
from jax.experimental.pallas import tpu as pltpu
from jax.experimental.pallas import tpu_sc as plsc

N_NODES = 10000
N_EDGES = 160000
IN_DIM = 128
D = 16

NC = 2
NS = 16
NW = NC * NS
CHUNK = 128
N_CHUNKS = N_EDGES // CHUNK
N_ROW_CHUNKS = N_NODES // CHUNK
ROW_TAIL = N_NODES - N_ROW_CHUNKS * CHUNK


def _proj_body(x_ref, w_ref, b_ref, o_ref):
    h = jnp.dot(x_ref[...], w_ref[...], preferred_element_type=jnp.float32)
    o_ref[...] = jnp.maximum(h + b_ref[...], 0.0)


def _project(x, W_in, b_in):
    return pl.pallas_call(
        _proj_body,
        grid=(10,),
        in_specs=[
            pl.BlockSpec((N_NODES // 10, IN_DIM), lambda i: (i, 0)),
            pl.BlockSpec((IN_DIM, D), lambda i: (0, 0)),
            pl.BlockSpec((1, D), lambda i: (0, 0)),
        ],
        out_specs=pl.BlockSpec((N_NODES // 10, D), lambda i: (i, 0)),
        out_shape=jax.ShapeDtypeStruct((N_NODES, D), jnp.float32),
    )(x, W_in, b_in.reshape(1, D))


def _softplus_body(r_ref, a_ref, o_ref):
    r = r_ref[...]
    sp = jnp.maximum(r, 0.0) + jnp.log1p(jnp.exp(-jnp.abs(r)))
    o_ref[...] = a_ref[0, 0] * sp


def _edge_weights(raw_w, alpha):
    w2 = pl.pallas_call(
        _softplus_body,
        in_specs=[
            pl.BlockSpec((2 * N_EDGES // 128, 128), lambda: (0, 0)),
            pl.BlockSpec((1, 1), lambda: (0, 0)),
        ],
        out_specs=pl.BlockSpec((2 * N_EDGES // 128, 128), lambda: (0, 0)),
        out_shape=jax.ShapeDtypeStruct((2 * N_EDGES // 128, 128), jnp.float32),
    )(raw_w.reshape(2 * N_EDGES // 128, 128), jnp.asarray(alpha, jnp.float32).reshape(1, 1))
    return w2.reshape(2 * N_EDGES)


_GDN = lax.GatherDimensionNumbers(
    offset_dims=(), collapsed_slice_dims=(0,), start_index_map=(0,)
)


def _vreg_gather(x, idx):
    return lax.gather(x, idx[:, None], _GDN, (1,),
                      mode=lax.GatherScatterMode.PROMISE_IN_BOUNDS)


_MESH = plsc.VectorSubcoreMesh(
    core_axis_name="c", subcore_axis_name="s", num_cores=NC, num_subcores=NS
)


@functools.partial(
    pl.kernel,
    out_type=jax.ShapeDtypeStruct((NC, N_NODES, 2 * D), jnp.float32),
    mesh=_MESH,
    compiler_params=pltpu.CompilerParams(
        needs_layout_passes=False, use_tc_tiling_on_sc=False
    ),
    scratch_types=[
        pltpu.VMEM((CHUNK,), jnp.int32),
        pltpu.VMEM((CHUNK,), jnp.int32),
        pltpu.VMEM((CHUNK, D), jnp.float32),
        pltpu.VMEM((CHUNK, D), jnp.float32),
        pltpu.VMEM((CHUNK, D, D), jnp.float32),
        pltpu.VMEM((CHUNK,), jnp.float32),
        pltpu.VMEM((CHUNK, 2 * D), jnp.float32),
        pltpu.SemaphoreType.DMA,
        pltpu.VMEM_SHARED((N_NODES, 2 * D), jnp.float32),
    ],
)
def _edge_kernel(h_hbm, ei_hbm, t_hbm, w_hbm,
                 out_hbm, sidx, didx, hsrc, hdst, tbuf, wbuf, msg, sem, acc):
    cid = lax.axis_index("c")
    sid = lax.axis_index("s")
    wid = sid * NC + cid

    zero16 = jnp.zeros((D,), jnp.float32)
    lanes = lax.iota(jnp.int32, D)
    one_hot = jnp.where(lanes == 0, 1.0, 0.0).astype(jnp.float32)

    def _zrow(c, carry):
        msg[c, pl.ds(0, D)] = zero16
        msg[c, pl.ds(D, D)] = zero16
        return carry

    lax.fori_loop(0, CHUNK, _zrow, 0)
    n_rc = jnp.where(sid < N_ROW_CHUNKS - (N_ROW_CHUNKS // NS) * NS,
                     N_ROW_CHUNKS // NS + 1, N_ROW_CHUNKS // NS)

    def _zchunk(t, carry):
        start = (sid + NS * t) * CHUNK
        pltpu.sync_copy(msg.at[pl.ds(0, CHUNK)], acc.at[pl.ds(start, CHUNK)])
        return carry

    lax.fori_loop(0, n_rc, _zchunk, 0)

    @pl.when(sid == NS - 1)
    def _ztail():
        pltpu.sync_copy(msg.at[pl.ds(0, ROW_TAIL)],
                        acc.at[pl.ds(N_ROW_CHUNKS * CHUNK, ROW_TAIL)])

    plsc.subcore_barrier()

    def _setrow(c, carry):
        msg[c, pl.ds(D, D)] = one_hot
        return carry

    lax.fori_loop(0, CHUNK, _setrow, 0)

    lanes16 = lanes * D
    _ROT = [(lanes + j) % D for j in range(D)]
    _DIAG = [lanes16 + _ROT[j] for j in range(D)]

    def _direction(xin, xsub, scat_idx, base2):
        pltpu.sync_copy(t_hbm.at[pl.ds(base2, CHUNK)], tbuf)
        pltpu.sync_copy(w_hbm.at[pl.ds(base2, CHUNK)], wbuf)

        def _edge(c, carry):
            cvec = jnp.full((D,), c, jnp.int32)
            xs = xin[c, :]
            acc4 = [zero16, zero16, zero16, zero16]
            for j in range(D):
                col = plsc.load_gather(tbuf, [cvec, lanes, _ROT[j]])
                xr = _vreg_gather(xs, _ROT[j])
                acc4[j % 4] = acc4[j % 4] + col * xr
            m = (acc4[0] + acc4[1]) + (acc4[2] + acc4[3]) - xsub[c, :]
            wv = plsc.load_gather(wbuf, [cvec])
            msg[c, pl.ds(0, D)] = wv * m
            return carry

        lax.fori_loop(0, CHUNK, _edge, 0)
        pltpu.sync_copy(msg, acc.at[scat_idx], add=True)

    n_chunks = jnp.where(wid < N_CHUNKS - (N_CHUNKS // NW) * NW, N_CHUNKS // NW + 1,
                         N_CHUNKS // NW)

    def _chunk(j, carry):
        base = (wid + j * NW) * CHUNK
        pltpu.sync_copy(ei_hbm.at[0, pl.ds(base, CHUNK)], sidx)
        pltpu.sync_copy(ei_hbm.at[1, pl.ds(base, CHUNK)], didx)
        pltpu.async_copy(h_hbm.at[sidx], hsrc, sem).wait()
        pltpu.async_copy(h_hbm.at[didx], hdst, sem).wait()
        _direction(hsrc, hdst, didx, base)
        _direction(hdst, hsrc, sidx, N_EDGES + base)
        return carry

    lax.fori_loop(0, n_chunks, _chunk, 0)
    plsc.subcore_barrier()

    def _dchunk(t, carry):
        start = (sid + NS * t) * CHUNK
        pltpu.sync_copy(acc.at[pl.ds(start, CHUNK)],
                        out_hbm.at[cid, pl.ds(start, CHUNK)])
        return carry

    lax.fori_loop(0, n_rc, _dchunk, 0)

    @pl.when(sid == NS - 1)
    def _dtail():
        pltpu.sync_copy(acc.at[pl.ds(N_ROW_CHUNKS * CHUNK, ROW_TAIL)],
                        out_hbm.at[cid, pl.ds(N_ROW_CHUNKS * CHUNK, ROW_TAIL)])


def _final_body(h_ref, a0_ref, a1_ref, g_ref, b_ref, o_ref):
    accs = a0_ref[...] + a1_ref[...]
    msgv = accs[:, 0:D]
    deg = accs[:, D:D + 1]
    out = h_ref[...] + msgv / jnp.maximum(deg, 1.0)
    mu = jnp.mean(out, axis=-1, keepdims=True)
    var = jnp.mean((out - mu) ** 2, axis=-1, keepdims=True)
    out = (out - mu) * lax.rsqrt(var + 1e-5) * g_ref[...] + b_ref[...]
    o_ref[...] = jnp.maximum(out, 0.0)


def _finalize(h, acc, ln_g, ln_b):
    blk = N_NODES // 10
    return pl.pallas_call(
        _final_body,
        grid=(10,),
        in_specs=[
            pl.BlockSpec((blk, D), lambda i: (i, 0)),
            pl.BlockSpec((blk, 2 * D), lambda i: (i, 0)),
            pl.BlockSpec((blk, 2 * D), lambda i: (i, 0)),
            pl.BlockSpec((1, D), lambda i: (0, 0)),
            pl.BlockSpec((1, D), lambda i: (0, 0)),
        ],
        out_specs=pl.BlockSpec((blk, D), lambda i: (i, 0)),
        out_shape=jax.ShapeDtypeStruct((N_NODES, D), jnp.float32),
    )(h, acc[0], acc[1], ln_g.reshape(1, D), ln_b.reshape(1, D))


def kernel(x, edge_index, W_in, b_in, T, raw_w, alpha, ln_g, ln_b):
    h = _project(x, W_in, b_in)
    w_all = _edge_weights(raw_w, alpha)
    acc = _edge_kernel(h, edge_index, T, w_all)
    return _finalize(h, acc, ln_g, ln_b)
